# Initial kernel scaffold; baseline (speedup 1.0000x reference)
import functools

import jax
import jax.numpy as jnp
from jax import lax
from jax.experimental import pallas as pl
from jax.experimental.pallas import tpu as pltpu
from jax.experimental.pallas import tpu_sc as plsc

N = 10000
NPAD = 10240
E = 320000
D = 128
DH = 64

NC = 2
NS = 16
L = 16

CH = 128
CPT = 157
NCHUNK = NS * CPT
EPAD = NCHUNK * CH
NST = NPAD // NS


def _dense_body(x_ref, th_ref, wt_ref, bt_ref, wh_ref, hraw_ref, gate_ref, base_ref):
    xb = x_ref[...]
    hraw_ref[...] = jnp.dot(xb, th_ref[...], preferred_element_type=jnp.float32)
    t = jnp.dot(xb, wt_ref[...], preferred_element_type=jnp.float32) + bt_ref[...]
    g = jax.nn.sigmoid(t)
    gate_ref[...] = g
    base_ref[...] = (1.0 - g) * jnp.dot(xb, wh_ref[...], preferred_element_type=jnp.float32)


def _dense_stage(xp, theta, W_t, b_t, W_h):
    bn = 256
    grid = (NPAD // bn,)
    return pl.pallas_call(
        _dense_body,
        grid=grid,
        in_specs=[
            pl.BlockSpec((bn, D), lambda i: (i, 0)),
            pl.BlockSpec((D, D), lambda i: (0, 0)),
            pl.BlockSpec((D, D), lambda i: (0, 0)),
            pl.BlockSpec((1, D), lambda i: (0, 0)),
            pl.BlockSpec((D, D), lambda i: (0, 0)),
        ],
        out_specs=[
            pl.BlockSpec((bn, D), lambda i: (i, 0)),
            pl.BlockSpec((bn, D), lambda i: (i, 0)),
            pl.BlockSpec((bn, D), lambda i: (i, 0)),
        ],
        out_shape=[
            jax.ShapeDtypeStruct((NPAD, D), jnp.float32),
            jax.ShapeDtypeStruct((NPAD, D), jnp.float32),
            jax.ShapeDtypeStruct((NPAD, D), jnp.float32),
        ],
    )(xp, theta, W_t, b_t.reshape(1, D), W_h)


def _rsqrt16(d):
    b = plsc.bitcast(d, jnp.int32)
    m = jnp.int32(0x5F3759DF) - lax.shift_right_logical(b, 1)
    y = plsc.bitcast(m, jnp.float32)
    for _ in range(3):
        y = y * (1.5 - 0.5 * d * y * y)
    return y


def _sc_body(row_hbm, col_hbm, hraw_hbm, norm_hbm, agg_hbm,
             sh_h, sh_agg, sh_deg,
             hbuf, gbuf, ridx, cidx, zvec2, zvec1, degb, nrmb, ones):
    c = lax.axis_index("c")
    s = lax.axis_index("s")
    r0 = s * NST
    k0 = s * CPT

    zv = jnp.zeros((L,), jnp.float32)
    for r in range(L):
        for f in range(DH // L):
            zvec2[r, pl.ds(f * L, L)] = zv
    for k in range(NST // L):
        zvec1[pl.ds(k * L, L)] = zv
    ov = jnp.full((L,), 1.0, jnp.float32)
    for k in range(CH // L):
        ones[pl.ds(k * L, L)] = ov
    for k in range(NST // L):
        pltpu.sync_copy(zvec2, sh_agg.at[pl.ds(r0 + k * L, L)])
    pltpu.sync_copy(zvec1, sh_deg.at[pl.ds(r0, NST)])

    pltpu.sync_copy(row_hbm.at[pl.ds(k0, CPT)], ridx)
    pltpu.sync_copy(col_hbm.at[pl.ds(k0, CPT)], cidx)

    plsc.subcore_barrier()

    @pl.loop(0, CPT)
    def _deg(k):
        pltpu.sync_copy(ones, sh_deg.at[ridx.at[k]], add=True)

    plsc.subcore_barrier()

    pltpu.sync_copy(sh_deg.at[pl.ds(r0, NST)], degb)
    for k in range(NST // L):
        d = degb[pl.ds(k * L, L)] + 1e-6
        nrmb[pl.ds(k * L, L)] = _rsqrt16(d)

    @pl.when(c == 0)
    def _():
        pltpu.sync_copy(nrmb, norm_hbm.at[pl.ds(r0, NST)])

    pltpu.sync_copy(hraw_hbm.at[pl.ds(r0, NST), pl.ds(c * DH, DH)], hbuf)

    lane = lax.iota(jnp.int32, (L,))

    @pl.loop(0, NST // L)
    def _scale(k):
        rows = k * L + lane
        nv = nrmb[pl.ds(k * L, L)]
        for f in range(DH):
            cols = jnp.zeros((L,), jnp.int32) + f
            v = plsc.load_gather(hbuf, [rows, cols])
            plsc.store_scatter(hbuf, [rows, cols], v * nv)

    pltpu.sync_copy(hbuf, sh_h.at[pl.ds(r0, NST)])

    plsc.subcore_barrier()

    @pl.loop(0, CPT)
    def _edges(k):
        pltpu.sync_copy(sh_h.at[cidx.at[k]], gbuf)
        pltpu.sync_copy(gbuf, sh_agg.at[ridx.at[k]], add=True)

    plsc.subcore_barrier()

    pltpu.sync_copy(sh_agg.at[pl.ds(r0, NST)],
                    agg_hbm.at[pl.ds(c * NPAD + r0, NST)])


def _sc_stage(row2d, col2d, hraw):
    mesh = plsc.VectorSubcoreMesh(core_axis_name="c", subcore_axis_name="s",
                                  num_cores=NC, num_subcores=NS)
    return pl.kernel(
        _sc_body,
        out_type=[
            jax.ShapeDtypeStruct((NPAD,), jnp.float32),
            jax.ShapeDtypeStruct((NC * NPAD, DH), jnp.float32),
        ],
        mesh=mesh,
        scratch_types=[
            pltpu.VMEM_SHARED((NPAD, DH), jnp.float32),
            pltpu.VMEM_SHARED((NPAD, DH), jnp.float32),
            pltpu.VMEM_SHARED((NPAD,), jnp.float32),
            pltpu.VMEM((NST, DH), jnp.float32),
            pltpu.VMEM((CH, DH), jnp.float32),
            pltpu.VMEM((CPT, CH), jnp.int32),
            pltpu.VMEM((CPT, CH), jnp.int32),
            pltpu.VMEM((L, DH), jnp.float32),
            pltpu.VMEM((NST,), jnp.float32),
            pltpu.VMEM((NST,), jnp.float32),
            pltpu.VMEM((NST,), jnp.float32),
            pltpu.VMEM((CH,), jnp.float32),
        ],
    )(row2d, col2d, hraw)


def _final_body(a0_ref, a1_ref, nrm_ref, gate_ref, base_ref, out_ref):
    agg = jnp.concatenate([a0_ref[...], a1_ref[...]], axis=1)
    out_ref[...] = jnp.maximum(
        gate_ref[...] * (agg * nrm_ref[...]) + base_ref[...], 0.0)


def _final_stage(a0, a1, norm, gate, base):
    bn = 256
    grid = (NPAD // bn,)
    return pl.pallas_call(
        _final_body,
        grid=grid,
        in_specs=[
            pl.BlockSpec((bn, DH), lambda i: (i, 0)),
            pl.BlockSpec((bn, DH), lambda i: (i, 0)),
            pl.BlockSpec((bn, 1), lambda i: (i, 0)),
            pl.BlockSpec((bn, D), lambda i: (i, 0)),
            pl.BlockSpec((bn, D), lambda i: (i, 0)),
        ],
        out_specs=pl.BlockSpec((bn, D), lambda i: (i, 0)),
        out_shape=jax.ShapeDtypeStruct((NPAD, D), jnp.float32),
    )(a0, a1, norm, gate, base)


@jax.jit
def kernel(x, edge_index, W_t, b_t, W_h, theta):
    xp = jnp.pad(x[0], ((0, NPAD - N), (0, 0)))
    row = edge_index[0].astype(jnp.int32)
    col = edge_index[1].astype(jnp.int32)
    rowp = jnp.concatenate([row, jnp.full((EPAD - E,), NPAD - 1, jnp.int32)])
    colp = jnp.concatenate([col, jnp.zeros((EPAD - E,), jnp.int32)])
    row2d = rowp.reshape(NCHUNK, CH)
    col2d = colp.reshape(NCHUNK, CH)

    hraw, gate, base = _dense_stage(xp, theta, W_t, b_t, W_h)
    norm, agg2 = _sc_stage(row2d, col2d, hraw)
    out = _final_stage(agg2[:NPAD], agg2[NPAD:], norm.reshape(NPAD, 1),
                       gate, base)
    return out[:N][None]


# speedup vs baseline: 5.5888x; 5.5888x over previous
import jax
import jax.numpy as jnp
from jax import lax
from jax.experimental import pallas as pl
from jax.experimental.pallas import tpu as pltpu
from jax.experimental.pallas import tpu_sc as plsc

N = 10000
NPAD = 10240
E = 320000
D = 128
DH = 64

NC = 2
NS = 16
L = 16

CH = 128
CPT = 160
NCHUNK = NS * CPT
EPAD = NCHUNK * CH
NST = NPAD // NS
CPT_DEG = NCHUNK // (NC * NS)
CG = 8


def _deg_body(row_hbm, deg_hbm, sh_deg, ridx, zvec1, ones):
    c = lax.axis_index("c")
    s = lax.axis_index("s")
    r0 = s * NST
    k0 = (c * NS + s) * CPT_DEG

    zv = jnp.zeros((L,), jnp.float32)
    for k in range(NST // L):
        zvec1[pl.ds(k * L, L)] = zv
    ov = jnp.full((L,), 1.0, jnp.float32)
    for k in range(CH // L):
        ones[pl.ds(k * L, L)] = ov
    pltpu.sync_copy(zvec1, sh_deg.at[pl.ds(r0, NST)])
    pltpu.sync_copy(row_hbm.at[pl.ds(k0, CPT_DEG)], ridx)

    plsc.subcore_barrier()

    @pl.loop(0, CPT_DEG)
    def _deg(k):
        pltpu.sync_copy(ones, sh_deg.at[ridx.at[k]], add=True)

    plsc.subcore_barrier()

    pltpu.sync_copy(sh_deg.at[pl.ds(r0, NST)],
                    deg_hbm.at[pl.ds(c * NPAD + r0, NST)])


def _deg_stage(row2d):
    mesh = plsc.VectorSubcoreMesh(core_axis_name="c", subcore_axis_name="s",
                                  num_cores=NC, num_subcores=NS)
    return pl.kernel(
        _deg_body,
        out_type=jax.ShapeDtypeStruct((NC * NPAD,), jnp.float32),
        mesh=mesh,
        compiler_params=pltpu.CompilerParams(use_tc_tiling_on_sc=False),
        scratch_types=[
            pltpu.VMEM_SHARED((NPAD,), jnp.float32),
            pltpu.VMEM((CPT_DEG, CH), jnp.int32),
            pltpu.VMEM((NST,), jnp.float32),
            pltpu.VMEM((CH,), jnp.float32),
        ],
    )(row2d)


def _dense_body(x_ref, d0_ref, d1_ref, th_ref, wt_ref, bt_ref, wh_ref,
                h0_ref, h1_ref, gn_ref, base_ref):
    xb = x_ref[...]
    norm = lax.rsqrt(d0_ref[...] + d1_ref[...] + 1e-6)
    h = jnp.dot(xb, th_ref[...], preferred_element_type=jnp.float32) * norm
    h0_ref[...] = h[:, :DH]
    h1_ref[...] = h[:, DH:]
    t = jnp.dot(xb, wt_ref[...], preferred_element_type=jnp.float32) + bt_ref[...]
    g = jax.nn.sigmoid(t)
    gn_ref[...] = g * norm
    base_ref[...] = (1.0 - g) * jnp.dot(xb, wh_ref[...], preferred_element_type=jnp.float32)


def _dense_stage(xp, d0, d1, theta, W_t, b_t, W_h):
    bn = 256
    grid = (NPAD // bn,)
    return pl.pallas_call(
        _dense_body,
        grid=grid,
        in_specs=[
            pl.BlockSpec((bn, D), lambda i: (i, 0)),
            pl.BlockSpec((bn, 1), lambda i: (i, 0)),
            pl.BlockSpec((bn, 1), lambda i: (i, 0)),
            pl.BlockSpec((D, D), lambda i: (0, 0)),
            pl.BlockSpec((D, D), lambda i: (0, 0)),
            pl.BlockSpec((1, D), lambda i: (0, 0)),
            pl.BlockSpec((D, D), lambda i: (0, 0)),
        ],
        out_specs=[
            pl.BlockSpec((bn, DH), lambda i: (i, 0)),
            pl.BlockSpec((bn, DH), lambda i: (i, 0)),
            pl.BlockSpec((bn, D), lambda i: (i, 0)),
            pl.BlockSpec((bn, D), lambda i: (i, 0)),
        ],
        out_shape=[
            jax.ShapeDtypeStruct((NPAD, DH), jnp.float32),
            jax.ShapeDtypeStruct((NPAD, DH), jnp.float32),
            jax.ShapeDtypeStruct((NPAD, D), jnp.float32),
            jax.ShapeDtypeStruct((NPAD, D), jnp.float32),
        ],
    )(xp, d0, d1, theta, W_t, b_t.reshape(1, D), W_h)


def _agg_body(row_hbm, col_hbm, h0_hbm, h1_hbm, agg_hbm,
              sh_h, sh_agg, gbuf, ridxb, cidxb, zvec2):
    c = lax.axis_index("c")
    s = lax.axis_index("s")
    r0 = s * NST
    k0 = s * CPT

    zv = jnp.zeros((L,), jnp.float32)
    for r in range(L):
        for f in range(DH // L):
            zvec2[r, pl.ds(f * L, L)] = zv
    for k in range(NST // L):
        pltpu.sync_copy(zvec2, sh_agg.at[pl.ds(r0 + k * L, L)])

    @pl.when(c == 0)
    def _():
        pltpu.sync_copy(h0_hbm.at[pl.ds(r0, NST)], sh_h.at[pl.ds(r0, NST)])

    @pl.when(c == 1)
    def _():
        pltpu.sync_copy(h1_hbm.at[pl.ds(r0, NST)], sh_h.at[pl.ds(r0, NST)])

    plsc.subcore_barrier()

    @pl.loop(0, CPT // CG)
    def _grp(g):
        pltpu.sync_copy(row_hbm.at[pl.ds(k0 + g * CG, CG)], ridxb)
        pltpu.sync_copy(col_hbm.at[pl.ds(k0 + g * CG, CG)], cidxb)
        for j in range(CG):
            pltpu.sync_copy(sh_h.at[cidxb.at[j]], gbuf)
            pltpu.sync_copy(gbuf, sh_agg.at[ridxb.at[j]], add=True)

    plsc.subcore_barrier()

    pltpu.sync_copy(sh_agg.at[pl.ds(r0, NST)],
                    agg_hbm.at[pl.ds(c * NPAD + r0, NST)])


def _agg_stage(row2d, col2d, h0, h1):
    mesh = plsc.VectorSubcoreMesh(core_axis_name="c", subcore_axis_name="s",
                                  num_cores=NC, num_subcores=NS)
    return pl.kernel(
        _agg_body,
        out_type=jax.ShapeDtypeStruct((NC * NPAD, DH), jnp.float32),
        mesh=mesh,
        compiler_params=pltpu.CompilerParams(use_tc_tiling_on_sc=False),
        scratch_types=[
            pltpu.VMEM_SHARED((NPAD, DH), jnp.float32),
            pltpu.VMEM_SHARED((NPAD, DH), jnp.float32),
            pltpu.VMEM((CH, DH), jnp.float32),
            pltpu.VMEM((CG, CH), jnp.int32),
            pltpu.VMEM((CG, CH), jnp.int32),
            pltpu.VMEM((L, DH), jnp.float32),
        ],
    )(row2d, col2d, h0, h1)


def _final_body(a0_ref, a1_ref, gn_ref, base_ref, out_ref):
    agg = jnp.concatenate([a0_ref[...], a1_ref[...]], axis=1)
    out_ref[...] = jnp.maximum(gn_ref[...] * agg + base_ref[...], 0.0)


def _final_stage(a0, a1, gn, base):
    bn = 256
    grid = (NPAD // bn,)
    return pl.pallas_call(
        _final_body,
        grid=grid,
        in_specs=[
            pl.BlockSpec((bn, DH), lambda i: (i, 0)),
            pl.BlockSpec((bn, DH), lambda i: (i, 0)),
            pl.BlockSpec((bn, D), lambda i: (i, 0)),
            pl.BlockSpec((bn, D), lambda i: (i, 0)),
        ],
        out_specs=pl.BlockSpec((bn, D), lambda i: (i, 0)),
        out_shape=jax.ShapeDtypeStruct((NPAD, D), jnp.float32),
    )(a0, a1, gn, base)


@jax.jit
def kernel(x, edge_index, W_t, b_t, W_h, theta):
    xp = jnp.pad(x[0], ((0, NPAD - N), (0, 0)))
    row = edge_index[0].astype(jnp.int32)
    col = edge_index[1].astype(jnp.int32)
    npad_e = EPAD - E
    rowp = jnp.concatenate(
        [row, N + (jnp.arange(npad_e, dtype=jnp.int32) % (NPAD - N))])
    colp = jnp.concatenate(
        [col, jnp.arange(npad_e, dtype=jnp.int32) % N])
    row2d = rowp.reshape(NCHUNK, CH)
    col2d = colp.reshape(NCHUNK, CH)

    deg = _deg_stage(row2d)
    d0 = deg[:NPAD].reshape(NPAD, 1)
    d1 = deg[NPAD:].reshape(NPAD, 1)
    h0, h1, gn, base = _dense_stage(xp, d0, d1, theta, W_t, b_t, W_h)
    agg2 = _agg_stage(row2d, col2d, h0, h1)
    out = _final_stage(agg2[:NPAD], agg2[NPAD:], gn, base)
    return out[:N][None]
